# baseline (speedup 1.0000x reference)
import functools

import jax
import jax.numpy as jnp
from jax import lax
from jax.experimental import pallas as pl
from jax.experimental.pallas import tpu as pltpu
from jax.experimental.pallas import tpu_sc as plsc

LANES = 16
NC, NS = 2, 16
NW = NC * NS
CHUNK = 32
NSLOT = 4


def _pre_body(seg_ref, pos_ref, pos0_ref, delta_ref):
    pos0_ref[...] = pos_ref[...] + seg_ref[0:1, :]
    delta_ref[...] = seg_ref[1:2, :] - seg_ref[0:1, :]


def _make_pre(seg, pos):
    t, d = seg.shape
    s = pos.shape[0]
    assert t == 2
    return pl.pallas_call(
        _pre_body,
        out_shape=(jax.ShapeDtypeStruct((s, d), jnp.float32),
                   jax.ShapeDtypeStruct((1, d), jnp.float32)),
    )(seg, pos)


def _sc_body(seq, total_rows, d,
             tok_hbm, pos0_hbm, delta_hbm, ids_hbm, tt_hbm, out_hbm,
             idxc, ttc, pos0blk, delta_v, bufs, sem_g, sem_w):
    vecs = d // LANES
    rows_per_w = total_rows // NW
    seqs_per_w = rows_per_w // seq
    pblocks = seq // CHUNK
    k_iters = seqs_per_w // NSLOT
    wid = lax.axis_index("s") * NC + lax.axis_index("c")
    wbase = wid * rows_per_w
    iota = lax.iota(jnp.int32, LANES)
    zero16 = iota * 0

    pltpu.sync_copy(delta_hbm, delta_v)

    def row_base(q, p):
        return wbase + q * seq + p * CHUNK

    def issue(slot, q, p):
        base = row_base(q, p)
        pltpu.sync_copy(ids_hbm.at[pl.ds(base, CHUNK)], idxc.at[slot])
        pltpu.sync_copy(tt_hbm.at[pl.ds(base, CHUNK)],
                        ttc.at[pl.ds(slot * CHUNK, CHUNK)])
        pltpu.async_copy(tok_hbm.at[idxc.at[slot]], bufs.at[slot], sem_g[slot])

    def wait_gather(slot):
        pltpu.make_async_copy(
            tok_hbm.at[pl.ds(0, CHUNK)], bufs.at[slot], sem_g[slot]).wait()

    def splat(v, l):
        idx = (zero16 + l)[:, None]
        dn = lax.GatherDimensionNumbers(
            offset_dims=(), collapsed_slice_dims=(0,), start_index_map=(0,))
        return lax.gather(v, idx, dn, slice_sizes=(1,),
                          mode=lax.GatherScatterMode.PROMISE_IN_BOUNDS)

    def vadd_pair(s0, s1):
        bufA = bufs.at[s0]
        bufB = bufs.at[s1]

        @pl.loop(0, CHUNK // LANES)
        def _grp(g):
            tfvA = ttc[pl.ds(s0 * CHUNK + g * LANES, LANES)].astype(
                jnp.float32)
            tfvB = ttc[pl.ds(s1 * CHUNK + g * LANES, LANES)].astype(
                jnp.float32)

            @pl.loop(0, LANES)
            def _lane(l):
                i = g * LANES + l
                tfA = splat(tfvA, l)
                tfB = splat(tfvB, l)
                for j in range(vecs):
                    sl = pl.ds(j * LANES, LANES)
                    pv = pos0blk[i, sl]
                    dv = delta_v[0, sl]
                    bufA[i, sl] = bufA[i, sl] + (pv + tfA * dv)
                    bufB[i, sl] = bufB[i, sl] + (pv + tfB * dv)

    def write(slot, q, p):
        base = row_base(q, p)
        pltpu.async_copy(bufs.at[slot], out_hbm.at[pl.ds(base, CHUNK)],
                         sem_w[slot])

    def wait_write(slot):
        pltpu.make_async_copy(
            bufs.at[slot], out_hbm.at[pl.ds(0, CHUNK)], sem_w[slot]).wait()

    @pl.loop(0, pblocks)
    def _pblock(p):
        pltpu.sync_copy(pos0_hbm.at[pl.ds(p * CHUNK, CHUNK)], pos0blk)
        for s in range(NSLOT):
            issue(s, s, p)

        @pl.loop(0, k_iters)
        def _k(k):
            for s0, s1 in ((0, 1), (2, 3)):
                wait_gather(s0)
                wait_gather(s1)
                vadd_pair(s0, s1)
                write(s0, k * NSLOT + s0, p)
                write(s1, k * NSLOT + s1, p)

            @pl.when(k < k_iters - 1)
            def _reissue():
                for s in range(NSLOT):
                    wait_write(s)
                    issue(s, (k + 1) * NSLOT + s, p)

        for s in range(NSLOT):
            wait_write(s)


@functools.lru_cache(maxsize=None)
def _make_sc(seq, total_rows, d):
    rows_per_w = total_rows // NW
    assert total_rows % NW == 0 and rows_per_w % seq == 0
    assert seq % CHUNK == 0 and (rows_per_w // seq) % NSLOT == 0
    assert d % LANES == 0
    mesh = plsc.VectorSubcoreMesh(
        core_axis_name="c", subcore_axis_name="s",
        num_cores=NC, num_subcores=NS)
    return pl.kernel(
        functools.partial(_sc_body, seq, total_rows, d),
        out_type=jax.ShapeDtypeStruct((total_rows, d), jnp.float32),
        mesh=mesh,
        scratch_types=[
            pltpu.VMEM((NSLOT, CHUNK), jnp.int32),
            pltpu.VMEM((NSLOT * CHUNK,), jnp.int32),
            pltpu.VMEM((CHUNK, d), jnp.float32),
            pltpu.VMEM((1, d), jnp.float32),
            pltpu.VMEM((NSLOT, CHUNK, d), jnp.float32),
            [pltpu.SemaphoreType.DMA] * NSLOT,
            [pltpu.SemaphoreType.DMA] * NSLOT,
        ],
    )


def kernel(input_ids, token_type_ids, token_embedding, segment_embedding,
           position_embedding):
    b, s = input_ids.shape
    d = token_embedding.shape[1]
    pos0, delta = _make_pre(segment_embedding, position_embedding)
    ids = input_ids.reshape(-1).astype(jnp.int32)
    tt = token_type_ids.reshape(-1).astype(jnp.int32)
    sc = _make_sc(s, b * s, d)
    out = sc(token_embedding, pos0, delta, ids, tt)
    return out.reshape(b, s, d)


# speedup vs baseline: 1.2775x; 1.2775x over previous
import functools

import jax
import jax.numpy as jnp
from jax import lax
from jax.experimental import pallas as pl
from jax.experimental.pallas import tpu as pltpu
from jax.experimental.pallas import tpu_sc as plsc

LANES = 16
NC, NS = 2, 16
NW = NC * NS
CHUNK = 32
NSLOT = 4


def _pre_body(seg_ref, pos_ref, pos0_ref, delta_ref):
    pos0_ref[...] = pos_ref[...] + seg_ref[0:1, :]
    delta_ref[...] = seg_ref[1:2, :] - seg_ref[0:1, :]


def _make_pre(seg, pos):
    t, d = seg.shape
    s = pos.shape[0]
    assert t == 2
    return pl.pallas_call(
        _pre_body,
        out_shape=(jax.ShapeDtypeStruct((s, d), jnp.float32),
                   jax.ShapeDtypeStruct((1, d), jnp.float32)),
    )(seg, pos)


def _sc_body(seq, total_rows, d,
             tok_hbm, pos0_hbm, delta_hbm, ids_hbm, tt_hbm, out_hbm,
             idxc, ttc, pos0blk, delta_v, bufs, sem_g, sem_w):
    vecs = d // LANES
    rows_per_w = total_rows // NW
    seqs_per_w = rows_per_w // seq
    pblocks = seq // CHUNK
    k_iters = seqs_per_w // NSLOT
    wid = lax.axis_index("s") * NC + lax.axis_index("c")
    wbase = wid * rows_per_w
    iota = lax.iota(jnp.int32, LANES)
    zero16 = iota * 0

    pltpu.sync_copy(delta_hbm, delta_v)

    def row_base(q, p):
        return wbase + q * seq + p * CHUNK

    def issue(slot, q, p):
        base = row_base(q, p)
        pltpu.sync_copy(ids_hbm.at[pl.ds(base, CHUNK)], idxc.at[slot])
        pltpu.sync_copy(tt_hbm.at[pl.ds(base, CHUNK)],
                        ttc.at[pl.ds(slot * CHUNK, CHUNK)])
        pltpu.async_copy(tok_hbm.at[idxc.at[slot]], bufs.at[slot], sem_g[slot])

    def wait_gather(slot):
        pltpu.make_async_copy(
            tok_hbm.at[pl.ds(0, CHUNK)], bufs.at[slot], sem_g[slot]).wait()

    def splat(v, l):
        idx = (zero16 + l)[:, None]
        dn = lax.GatherDimensionNumbers(
            offset_dims=(), collapsed_slice_dims=(0,), start_index_map=(0,))
        return lax.gather(v, idx, dn, slice_sizes=(1,),
                          mode=lax.GatherScatterMode.PROMISE_IN_BOUNDS)

    def vadd(slot):
        buf = bufs.at[slot]

        @pl.loop(0, CHUNK // LANES)
        def _grp(g):
            tfv = ttc[pl.ds(slot * CHUNK + g * LANES, LANES)].astype(
                jnp.float32)

            @plsc.parallel_loop(0, LANES)
            def _lane(l):
                i = g * LANES + l
                tf = splat(tfv, l)
                for j in range(vecs):
                    sl = pl.ds(j * LANES, LANES)
                    buf[i, sl] = buf[i, sl] + (pos0blk[i, sl]
                                               + tf * delta_v[0, sl])

    def write(slot, q, p):
        base = row_base(q, p)
        pltpu.async_copy(bufs.at[slot], out_hbm.at[pl.ds(base, CHUNK)],
                         sem_w[slot])

    def wait_write(slot):
        pltpu.make_async_copy(
            bufs.at[slot], out_hbm.at[pl.ds(0, CHUNK)], sem_w[slot]).wait()

    @pl.loop(0, pblocks)
    def _pblock(p):
        pltpu.sync_copy(pos0_hbm.at[pl.ds(p * CHUNK, CHUNK)], pos0blk)
        for s in range(NSLOT):
            issue(s, s, p)

        @pl.loop(0, k_iters)
        def _k(k):
            for s in range(NSLOT):
                wait_gather(s)
                vadd(s)
                write(s, k * NSLOT + s, p)

            @pl.when(k < k_iters - 1)
            def _reissue():
                for s in range(NSLOT):
                    wait_write(s)
                    issue(s, (k + 1) * NSLOT + s, p)

        for s in range(NSLOT):
            wait_write(s)


@functools.lru_cache(maxsize=None)
def _make_sc(seq, total_rows, d):
    rows_per_w = total_rows // NW
    assert total_rows % NW == 0 and rows_per_w % seq == 0
    assert seq % CHUNK == 0 and (rows_per_w // seq) % NSLOT == 0
    assert d % LANES == 0
    mesh = plsc.VectorSubcoreMesh(
        core_axis_name="c", subcore_axis_name="s",
        num_cores=NC, num_subcores=NS)
    return pl.kernel(
        functools.partial(_sc_body, seq, total_rows, d),
        out_type=jax.ShapeDtypeStruct((total_rows, d), jnp.float32),
        mesh=mesh,
        scratch_types=[
            pltpu.VMEM((NSLOT, CHUNK), jnp.int32),
            pltpu.VMEM((NSLOT * CHUNK,), jnp.int32),
            pltpu.VMEM((CHUNK, d), jnp.float32),
            pltpu.VMEM((1, d), jnp.float32),
            pltpu.VMEM((NSLOT, CHUNK, d), jnp.float32),
            [pltpu.SemaphoreType.DMA] * NSLOT,
            [pltpu.SemaphoreType.DMA] * NSLOT,
        ],
    )


def kernel(input_ids, token_type_ids, token_embedding, segment_embedding,
           position_embedding):
    b, s = input_ids.shape
    d = token_embedding.shape[1]
    pos0, delta = _make_pre(segment_embedding, position_embedding)
    ids = input_ids.reshape(-1).astype(jnp.int32)
    tt = token_type_ids.reshape(-1).astype(jnp.int32)
    sc = _make_sc(s, b * s, d)
    out = sc(token_embedding, pos0, delta, ids, tt)
    return out.reshape(b, s, d)


# speedup vs baseline: 2.2103x; 1.7302x over previous
import functools

import jax
import jax.numpy as jnp
from jax import lax
from jax.experimental import pallas as pl
from jax.experimental.pallas import tpu as pltpu
from jax.experimental.pallas import tpu_sc as plsc

LANES = 16
NC, NS = 2, 16
NW = NC * NS
CHUNK = 32
NSLOT = 4


def _pre_body(seg_ref, pos_ref, pos0_ref, delta_ref):
    pos0_ref[...] = pos_ref[...] + seg_ref[0:1, :]
    delta_ref[...] = seg_ref[1:2, :] - seg_ref[0:1, :]


def _make_pre(seg, pos):
    t, d = seg.shape
    s = pos.shape[0]
    assert t == 2
    return pl.pallas_call(
        _pre_body,
        out_shape=(jax.ShapeDtypeStruct((s, d), jnp.float32),
                   jax.ShapeDtypeStruct((1, d), jnp.float32)),
    )(seg, pos)


def _sc_body(seq, total_rows, d,
             tok_hbm, pos0_hbm, delta_hbm, ids_hbm, tt_hbm, out_hbm,
             idxc, ttc, pos0blk, delta_v, buf0, buf1, buf2, buf3,
             sem_g, sem_w):
    bufs = (buf0, buf1, buf2, buf3)
    vecs = d // LANES
    rows_per_w = total_rows // NW
    seqs_per_w = rows_per_w // seq
    pblocks = seq // CHUNK
    k_iters = seqs_per_w // NSLOT
    wid = lax.axis_index("s") * NC + lax.axis_index("c")
    wbase = wid * rows_per_w
    iota = lax.iota(jnp.int32, LANES)
    zero16 = iota * 0

    pltpu.sync_copy(delta_hbm, delta_v)
    dvecs = [delta_v[0, pl.ds(j * LANES, LANES)] for j in range(vecs)]

    def row_base(q, p):
        return wbase + q * seq + p * CHUNK

    def issue(slot, q, p):
        base = row_base(q, p)
        pltpu.sync_copy(ids_hbm.at[pl.ds(base, CHUNK)], idxc.at[slot])
        pltpu.sync_copy(tt_hbm.at[pl.ds(base, CHUNK)],
                        ttc.at[pl.ds(slot * CHUNK, CHUNK)])
        pltpu.async_copy(tok_hbm.at[idxc.at[slot]], bufs[slot], sem_g[slot])

    def wait_gather(slot):
        pltpu.make_async_copy(
            tok_hbm.at[pl.ds(0, CHUNK)], bufs[slot], sem_g[slot]).wait()

    def splat(v, l):
        idx = (zero16 + l)[:, None]
        dn = lax.GatherDimensionNumbers(
            offset_dims=(), collapsed_slice_dims=(0,), start_index_map=(0,))
        return lax.gather(v, idx, dn, slice_sizes=(1,),
                          mode=lax.GatherScatterMode.PROMISE_IN_BOUNDS)

    def vadd_pair(s0, s1):
        bufA = bufs[s0]
        bufB = bufs[s1]

        @pl.loop(0, CHUNK // LANES)
        def _grp(g):
            tfvA = ttc[pl.ds(s0 * CHUNK + g * LANES, LANES)].astype(
                jnp.float32)
            tfvB = ttc[pl.ds(s1 * CHUNK + g * LANES, LANES)].astype(
                jnp.float32)

            @pl.loop(0, LANES)
            def _lane(l):
                i = g * LANES + l
                tfA = splat(tfvA, l)
                tfB = splat(tfvB, l)
                for j in range(vecs):
                    sl = pl.ds(j * LANES, LANES)
                    pv = pos0blk[i, sl]
                    bufA[i, sl] = bufA[i, sl] + (pv + tfA * dvecs[j])
                    bufB[i, sl] = bufB[i, sl] + (pv + tfB * dvecs[j])

    def write(slot, q, p):
        base = row_base(q, p)
        pltpu.async_copy(bufs[slot], out_hbm.at[pl.ds(base, CHUNK)],
                         sem_w[slot])

    def wait_write(slot):
        pltpu.make_async_copy(
            bufs[slot], out_hbm.at[pl.ds(0, CHUNK)], sem_w[slot]).wait()

    @pl.loop(0, pblocks)
    def _pblock(p):
        pltpu.sync_copy(pos0_hbm.at[pl.ds(p * CHUNK, CHUNK)], pos0blk)
        for s in range(NSLOT):
            issue(s, s, p)

        @pl.loop(0, k_iters)
        def _k(k):
            for s0, s1 in ((0, 1), (2, 3)):
                wait_gather(s0)
                wait_gather(s1)
                vadd_pair(s0, s1)
                write(s0, k * NSLOT + s0, p)
                write(s1, k * NSLOT + s1, p)

            @pl.when(k < k_iters - 1)
            def _reissue():
                for s in range(NSLOT):
                    wait_write(s)
                    issue(s, (k + 1) * NSLOT + s, p)

        for s in range(NSLOT):
            wait_write(s)


@functools.lru_cache(maxsize=None)
def _make_sc(seq, total_rows, d):
    rows_per_w = total_rows // NW
    assert total_rows % NW == 0 and rows_per_w % seq == 0
    assert seq % CHUNK == 0 and (rows_per_w // seq) % NSLOT == 0
    assert d % LANES == 0
    mesh = plsc.VectorSubcoreMesh(
        core_axis_name="c", subcore_axis_name="s",
        num_cores=NC, num_subcores=NS)
    return pl.kernel(
        functools.partial(_sc_body, seq, total_rows, d),
        out_type=jax.ShapeDtypeStruct((total_rows, d), jnp.float32),
        mesh=mesh,
        scratch_types=[
            pltpu.VMEM((NSLOT, CHUNK), jnp.int32),
            pltpu.VMEM((NSLOT * CHUNK,), jnp.int32),
            pltpu.VMEM((CHUNK, d), jnp.float32),
            pltpu.VMEM((1, d), jnp.float32),
            pltpu.VMEM((CHUNK, d), jnp.float32),
            pltpu.VMEM((CHUNK, d), jnp.float32),
            pltpu.VMEM((CHUNK, d), jnp.float32),
            pltpu.VMEM((CHUNK, d), jnp.float32),
            [pltpu.SemaphoreType.DMA] * NSLOT,
            [pltpu.SemaphoreType.DMA] * NSLOT,
        ],
    )


def kernel(input_ids, token_type_ids, token_embedding, segment_embedding,
           position_embedding):
    b, s = input_ids.shape
    d = token_embedding.shape[1]
    pos0, delta = _make_pre(segment_embedding, position_embedding)
    ids = input_ids.reshape(-1).astype(jnp.int32)
    tt = token_type_ids.reshape(-1).astype(jnp.int32)
    sc = _make_sc(s, b * s, d)
    out = sc(token_embedding, pos0, delta, ids, tt)
    return out.reshape(b, s, d)


# speedup vs baseline: 2.4051x; 1.0881x over previous
import functools

import jax
import jax.numpy as jnp
from jax import lax
from jax.experimental import pallas as pl
from jax.experimental.pallas import tpu as pltpu
from jax.experimental.pallas import tpu_sc as plsc

LANES = 16
NC, NS = 2, 16
NW = NC * NS
CHUNK = 32
SUP = 128
NSLOT = 4


def _pre_body(seg_ref, pos_ref, ids_ref, tt_ref, pos0_ref, delta_ref,
              comb_ref):
    pos0_ref[...] = pos_ref[...] + seg_ref[0:1, :]
    delta_ref[...] = seg_ref[1:2, :] - seg_ref[0:1, :]
    comb_ref[...] = ids_ref[...] * 2 + tt_ref[...]


def _make_pre(seg, pos, ids, tt):
    t, d = seg.shape
    s = pos.shape[0]
    assert t == 2
    return pl.pallas_call(
        _pre_body,
        out_shape=(jax.ShapeDtypeStruct((s, d), jnp.float32),
                   jax.ShapeDtypeStruct((1, d), jnp.float32),
                   jax.ShapeDtypeStruct(ids.shape, jnp.int32)),
    )(seg, pos, ids, tt)


def _sc_body(seq, total_rows, d,
             tok_hbm, pos0_hbm, delta_hbm, comb_hbm, out_hbm,
             idlist, cblk, idxc, pos0blk, delta_v, buf0, buf1, buf2, buf3,
             sem_i, sem_g, sem_w):
    bufs = (buf0, buf1, buf2, buf3)
    vecs = d // LANES
    rows_per_w = total_rows // NW
    seqs_per_w = rows_per_w // seq
    nsup = seq // SUP
    sub_blocks = SUP // CHUNK
    k_iters = seqs_per_w // NSLOT
    wid = lax.axis_index("s") * NC + lax.axis_index("c")
    wbase = wid * rows_per_w
    wrow0 = wid * (rows_per_w // SUP)
    iota = lax.iota(jnp.int32, LANES)
    zero16 = iota * 0

    pltpu.sync_copy(delta_hbm, delta_v)
    dvecs = [delta_v[0, pl.ds(j * LANES, LANES)] for j in range(vecs)]

    def row_base(q, p):
        return wbase + q * seq + p * CHUNK

    def issue(slot, q, p2):
        for v in range(CHUNK // LANES):
            cv = cblk[q, pl.ds(p2 * CHUNK + v * LANES, LANES)]
            idxc[slot, pl.ds(v * LANES, LANES)] = lax.shift_right_logical(
                cv, 1)
        pltpu.async_copy(tok_hbm.at[idxc.at[slot]], bufs[slot], sem_g[slot])

    def wait_gather(slot):
        pltpu.make_async_copy(
            tok_hbm.at[pl.ds(0, CHUNK)], bufs[slot], sem_g[slot]).wait()

    def splat(v, l):
        idx = (zero16 + l)[:, None]
        dn = lax.GatherDimensionNumbers(
            offset_dims=(), collapsed_slice_dims=(0,), start_index_map=(0,))
        return lax.gather(v, idx, dn, slice_sizes=(1,),
                          mode=lax.GatherScatterMode.PROMISE_IN_BOUNDS)

    def tf_vec(q, p2, g):
        cv = cblk[q, pl.ds(p2 * CHUNK + g * LANES, LANES)]
        return (cv & 1).astype(jnp.float32)

    def vadd_pair(s0, q0, s1, q1, p2):
        bufA = bufs[s0]
        bufB = bufs[s1]

        @pl.loop(0, CHUNK // LANES)
        def _grp(g):
            tfvA = tf_vec(q0, p2, g)
            tfvB = tf_vec(q1, p2, g)

            @pl.loop(0, LANES)
            def _lane(l):
                i = g * LANES + l
                tfA = splat(tfvA, l)
                tfB = splat(tfvB, l)
                for j in range(vecs):
                    sl = pl.ds(j * LANES, LANES)
                    pv = pos0blk[i, sl]
                    bufA[i, sl] = bufA[i, sl] + (pv + tfA * dvecs[j])
                    bufB[i, sl] = bufB[i, sl] + (pv + tfB * dvecs[j])

    def write(slot, q, p):
        base = row_base(q, p)
        pltpu.async_copy(bufs[slot], out_hbm.at[pl.ds(base, CHUNK)],
                         sem_w[slot])

    def wait_write(slot):
        pltpu.make_async_copy(
            bufs[slot], out_hbm.at[pl.ds(0, CHUNK)], sem_w[slot]).wait()

    @pl.loop(0, nsup)
    def _sup(sup):
        for j in range(seqs_per_w // LANES):
            idlist[pl.ds(j * LANES, LANES)] = (
                wrow0 + (iota + j * LANES) * nsup + sup)
        pltpu.async_copy(comb_hbm.at[idlist], cblk, sem_i).wait()

        @pl.loop(0, sub_blocks)
        def _pblock(p2):
            p = sup * sub_blocks + p2
            pltpu.sync_copy(pos0_hbm.at[pl.ds(p * CHUNK, CHUNK)], pos0blk)
            for s in range(NSLOT):
                issue(s, s, p2)

            @pl.loop(0, k_iters)
            def _k(k):
                for s0, s1 in ((0, 1), (2, 3)):
                    q0 = k * NSLOT + s0
                    q1 = k * NSLOT + s1
                    wait_gather(s0)
                    wait_gather(s1)
                    vadd_pair(s0, q0, s1, q1, p2)
                    write(s0, q0, p)
                    write(s1, q1, p)

                @pl.when(k < k_iters - 1)
                def _reissue():
                    for s in range(NSLOT):
                        wait_write(s)
                        issue(s, (k + 1) * NSLOT + s, p2)

            for s in range(NSLOT):
                wait_write(s)


@functools.lru_cache(maxsize=None)
def _make_sc(seq, total_rows, d):
    rows_per_w = total_rows // NW
    seqs_per_w = rows_per_w // seq
    assert total_rows % NW == 0 and rows_per_w % seq == 0
    assert seq % SUP == 0 and SUP % CHUNK == 0
    assert seqs_per_w % NSLOT == 0 and seqs_per_w % LANES == 0
    assert d % LANES == 0
    mesh = plsc.VectorSubcoreMesh(
        core_axis_name="c", subcore_axis_name="s",
        num_cores=NC, num_subcores=NS)
    return pl.kernel(
        functools.partial(_sc_body, seq, total_rows, d),
        out_type=jax.ShapeDtypeStruct((total_rows, d), jnp.float32),
        mesh=mesh,
        scratch_types=[
            pltpu.VMEM((seqs_per_w,), jnp.int32),
            pltpu.VMEM((seqs_per_w, SUP), jnp.int32),
            pltpu.VMEM((NSLOT, CHUNK), jnp.int32),
            pltpu.VMEM((CHUNK, d), jnp.float32),
            pltpu.VMEM((1, d), jnp.float32),
            pltpu.VMEM((CHUNK, d), jnp.float32),
            pltpu.VMEM((CHUNK, d), jnp.float32),
            pltpu.VMEM((CHUNK, d), jnp.float32),
            pltpu.VMEM((CHUNK, d), jnp.float32),
            pltpu.SemaphoreType.DMA,
            [pltpu.SemaphoreType.DMA] * NSLOT,
            [pltpu.SemaphoreType.DMA] * NSLOT,
        ],
    )


def kernel(input_ids, token_type_ids, token_embedding, segment_embedding,
           position_embedding):
    b, s = input_ids.shape
    d = token_embedding.shape[1]
    pos0, delta, comb = _make_pre(
        segment_embedding, position_embedding,
        input_ids.astype(jnp.int32), token_type_ids.astype(jnp.int32))
    comb2 = comb.reshape(-1, SUP)
    sc = _make_sc(s, b * s, d)
    out = sc(token_embedding, pos0, delta, comb2)
    return out.reshape(b, s, d)
